# baseline (speedup 1.0000x reference)
import jax
import jax.numpy as jnp
from jax import lax
from jax.experimental import pallas as pl
from jax.experimental.pallas import tpu as pltpu
from jax.experimental.pallas import tpu_sc as plsc

_H = 8
_P = 16
_R = 0.08
_G = 576
_DH = 32
_CH = 256
_TT2 = 256
_TSC = 2304


def _dot(a, b, dims):
    return jax.lax.dot_general(a, b, (dims, ((), ())),
                               preferred_element_type=jnp.float32)


def _prep_body(q_ref, fm_ref, rxy_ref, wv_ref, wcat_ref, bcat_ref,
               xf_ref, yf_ref, w_ref, tbl_ref):
    q = q_ref[0]
    proj = _dot(wcat_ref[0], q, ((1,), (1,))) + bcat_ref[0]
    offx = proj[0:16, :]
    offy = proj[16:32, :]
    wl = proj[32:48, :]
    wl = wl - jnp.max(wl, axis=0, keepdims=True)
    we = jnp.exp(wl)
    w_ref[0] = we / jnp.sum(we, axis=0, keepdims=True)
    rx = rxy_ref[0, 0:1, :]
    ry = rxy_ref[0, 1:2, :]
    xf_ref[0] = (rx + _R * offx) * 23.0
    yf_ref[0] = (ry + _R * offy) * 23.0
    tbl_ref[0] = _dot(fm_ref[0], wv_ref[0], ((0,), (1,)))


def _prep(q, fm, rxy, wv, wcat, bcat, B, T, D, C):
    return pl.pallas_call(
        _prep_body,
        grid=(B, _H),
        in_specs=[
            pl.BlockSpec((1, T, D), lambda b, h: (b, 0, 0)),
            pl.BlockSpec((1, C, _G), lambda b, h: (b, 0, 0)),
            pl.BlockSpec((1, 2, T), lambda b, h: (b, 0, 0)),
            pl.BlockSpec((1, _DH, C), lambda b, h: (h, 0, 0)),
            pl.BlockSpec((1, 48, D), lambda b, h: (h, 0, 0)),
            pl.BlockSpec((1, 48, 1), lambda b, h: (h, 0, 0)),
        ],
        out_specs=[
            pl.BlockSpec((1, _P, T), lambda b, h: (b * _H + h, 0, 0)),
            pl.BlockSpec((1, _P, T), lambda b, h: (b * _H + h, 0, 0)),
            pl.BlockSpec((1, _P, T), lambda b, h: (b * _H + h, 0, 0)),
            pl.BlockSpec((1, _G, _DH), lambda b, h: (b * _H + h, 0, 0)),
        ],
        out_shape=[
            jax.ShapeDtypeStruct((B * _H, _P, T), jnp.float32),
            jax.ShapeDtypeStruct((B * _H, _P, T), jnp.float32),
            jax.ShapeDtypeStruct((B * _H, _P, T), jnp.float32),
            jax.ShapeDtypeStruct((B * _H, _G, _DH), jnp.float32),
        ],
    )(q, fm, rxy, wv, wcat, bcat)


def _sc_body(xf_hbm, yf_hbm, w_hbm, tbl_hbm, out_hbm,
             tblb, xfb, yfb, wb, outb):
    wid = lax.axis_index("s") * 2 + lax.axis_index("c")
    pltpu.sync_copy(tbl_hbm.at[wid], tblb)

    def chunk_body(i, _):
        base = i * _CH
        pltpu.sync_copy(xf_hbm.at[wid, :, pl.ds(base, _CH)], xfb)
        pltpu.sync_copy(yf_hbm.at[wid, :, pl.ds(base, _CH)], yfb)
        pltpu.sync_copy(w_hbm.at[wid, :, pl.ds(base, _CH)], wb)

        def g_body(g, _):
            sl = pl.ds(g * 16, 16)
            for blk in range(_DH // 16):
                zeros16 = tuple(jnp.zeros((16,), jnp.float32) for _ in range(16))

                @plsc.parallel_loop(0, _P, carry=zeros16)
                def p_body(p, acc):
                    xfv = xfb[p, sl]
                    yfv = yfb[p, sl]
                    wv = wb[p, sl]
                    xt = xfv.astype(jnp.int32)
                    yt = yfv.astype(jnp.int32)
                    x0 = xt - jnp.where(xt.astype(jnp.float32) > xfv, 1, 0)
                    y0 = yt - jnp.where(yt.astype(jnp.float32) > yfv, 1, 0)
                    valid = ((xfv > -1.0) & (xfv < 24.0)
                             & (yfv > -1.0) & (yfv < 24.0))
                    wv = jnp.where(valid, wv, jnp.zeros((16,), jnp.float32))
                    x0 = jnp.clip(x0, -1, 24)
                    y0 = jnp.clip(y0, -1, 24)
                    fx = xfv - x0.astype(jnp.float32)
                    fy = yfv - y0.astype(jnp.float32)
                    wy1 = wv * fy
                    wy0 = wv - wy1
                    c01 = wy0 * fx
                    c00 = wy0 - c01
                    c11 = wy1 * fx
                    c10 = wy1 - c11
                    a0 = ((y0 + 1) * 26 + (x0 + 1)) * 33 + (blk * 16)
                    new = []
                    for c in range(16):
                        r0 = plsc.load_gather(tblb, [a0 + c])
                        r1 = plsc.load_gather(tblb, [a0 + (26 * 33 + c)])
                        lo0 = plsc.bitcast(r0 << 16, jnp.float32)
                        hi0 = plsc.bitcast(r0, jnp.float32)
                        lo1 = plsc.bitcast(r1 << 16, jnp.float32)
                        hi1 = plsc.bitcast(r1, jnp.float32)
                        v = (c00 * lo0 + c01 * hi0) + (c10 * lo1 + c11 * hi1)
                        new.append(acc[c] + v)
                    return tuple(new)

                for c in range(16):
                    outb[blk * 16 + c, sl] = p_body[c]
            return 0

        lax.fori_loop(0, _CH // 16, g_body, 0)
        pltpu.sync_copy(outb, out_hbm.at[wid, :, pl.ds(base, _CH)])
        return 0

    lax.fori_loop(0, _TSC // _CH, chunk_body, 0)


def _sc_sample(xf, yf, w, tbl, BH, T):
    mesh = plsc.VectorSubcoreMesh(core_axis_name="c", subcore_axis_name="s")
    return pl.kernel(
        _sc_body,
        out_type=jax.ShapeDtypeStruct((BH, _DH, T), jnp.float32),
        mesh=mesh,
        compiler_params=pltpu.CompilerParams(needs_layout_passes=False),
        scratch_types=[
            pltpu.VMEM((27 * 26 * 33,), jnp.int32),
            pltpu.VMEM((_P, _CH), jnp.float32),
            pltpu.VMEM((_P, _CH), jnp.float32),
            pltpu.VMEM((_P, _CH), jnp.float32),
            pltpu.VMEM((_DH, _CH), jnp.float32),
        ],
    )(xf, yf, w, tbl)


def _out_body(ctx_ref, wout_ref, bout_ref, out_ref):
    h = pl.program_id(2)
    contrib = _dot(ctx_ref[0], wout_ref[0], ((0,), (0,)))

    @pl.when(h == 0)
    def _():
        out_ref[0] = contrib + bout_ref[...]

    @pl.when(h != 0)
    def _():
        out_ref[0] += contrib


def _outproj(ctx, wout, bout, B, T, D):
    return pl.pallas_call(
        _out_body,
        grid=(B, T // _TT2, _H),
        in_specs=[
            pl.BlockSpec((1, _DH, _TT2), lambda b, t, h: (b * _H + h, 0, t)),
            pl.BlockSpec((1, _DH, D), lambda b, t, h: (h, 0, 0)),
            pl.BlockSpec((1, D), lambda b, t, h: (0, 0)),
        ],
        out_specs=pl.BlockSpec((1, _TT2, D), lambda b, t, h: (b, t, 0)),
        out_shape=jax.ShapeDtypeStruct((B, T, D), jnp.float32),
    )(ctx, wout, bout)


_TTC = 1792


def _tent_body(q_ref, fm_ref, rxy_ref, wv_ref, woff_ref, boff_ref, ww_ref,
               bw_ref, wout_ref, bout_ref, out_ref, vs_ref):
    t_idx = pl.program_id(1)
    h = pl.program_id(2)

    @pl.when(t_idx == 0)
    def _():
        vs_ref[h] = _dot(wv_ref[0], fm_ref[0], ((1,), (0,)))

    q = q_ref[0]
    off = _dot(woff_ref[0], q, ((1,), (1,))) + boff_ref[0]
    wl = _dot(ww_ref[0], q, ((1,), (1,))) + bw_ref[0]
    wl = wl - jnp.max(wl, axis=0, keepdims=True)
    we = jnp.exp(wl)
    w = we / jnp.sum(we, axis=0, keepdims=True)

    rx = rxy_ref[0, 0:1, :]
    ry = rxy_ref[0, 1:2, :]
    xs = jax.lax.broadcasted_iota(jnp.int32, (24, 1), 0).astype(jnp.float32)

    acc = jnp.zeros((24, 24, _TTC), jnp.float32)
    for p in range(_P):
        xf = (rx + _R * off[2 * p:2 * p + 1, :]) * 23.0
        yf = (ry + _R * off[2 * p + 1:2 * p + 2, :]) * 23.0
        tx = jnp.maximum(1.0 - jnp.abs(xs - xf), 0.0)
        ty = jnp.maximum(1.0 - jnp.abs(xs - yf), 0.0)
        wtx = w[p:p + 1, :] * tx
        acc = acc + wtx[None, :, :] * ty[:, None, :]
    acc = acc.reshape(_G, _TTC)

    ctx = _dot(vs_ref[h], acc, ((1,), (0,)))
    contrib = _dot(ctx, wout_ref[0], ((0,), (0,)))

    @pl.when(h == 0)
    def _():
        out_ref[0] = contrib + bout_ref[...]

    @pl.when(h != 0)
    def _():
        out_ref[0] += contrib


def _tent(q2, fm, rxy2, wv, woff, boff, ww, bw, wout, bout, B, T2, D, C):
    return pl.pallas_call(
        _tent_body,
        grid=(B, T2 // _TTC, _H),
        in_specs=[
            pl.BlockSpec((1, _TTC, D), lambda b, t, h: (b, t, 0)),
            pl.BlockSpec((1, C, _G), lambda b, t, h: (b, 0, 0)),
            pl.BlockSpec((1, 2, _TTC), lambda b, t, h: (b, 0, t)),
            pl.BlockSpec((1, _DH, C), lambda b, t, h: (h, 0, 0)),
            pl.BlockSpec((1, 2 * _P, D), lambda b, t, h: (h, 0, 0)),
            pl.BlockSpec((1, 2 * _P, 1), lambda b, t, h: (h, 0, 0)),
            pl.BlockSpec((1, _P, D), lambda b, t, h: (h, 0, 0)),
            pl.BlockSpec((1, _P, 1), lambda b, t, h: (h, 0, 0)),
            pl.BlockSpec((1, _DH, D), lambda b, t, h: (h, 0, 0)),
            pl.BlockSpec((1, D), lambda b, t, h: (0, 0)),
        ],
        out_specs=pl.BlockSpec((1, _TTC, D), lambda b, t, h: (b, t, 0)),
        out_shape=jax.ShapeDtypeStruct((B, T2, D), jnp.float32),
        scratch_shapes=[pltpu.VMEM((_H, _DH, _G), jnp.float32)],
    )(q2, fm, rxy2, wv, woff, boff, ww, bw, wout, bout)


def kernel(q, fmap, ref_xy, W_v, W_off, b_off, W_w, b_w, W_out, b_out):
    B, T, D = q.shape
    C = fmap.shape[1]
    fm = fmap.reshape(B, C, _G)
    rxy = ref_xy.transpose(0, 2, 1)
    wv = W_v.reshape(_H, _DH, C)
    woff4 = W_off.reshape(_H, _P, 2, D)
    wcat = jnp.concatenate(
        [woff4[:, :, 0, :], woff4[:, :, 1, :], W_w.reshape(_H, _P, D)], axis=1)
    boff4 = b_off.reshape(_H, _P, 2)
    bcat = jnp.concatenate(
        [boff4[:, :, 0], boff4[:, :, 1], b_w.reshape(_H, _P)],
        axis=1).reshape(_H, 48, 1)
    wout = W_out.T.reshape(_H, _DH, D)
    bout = b_out.reshape(1, D)

    q_sc = q[:, :_TSC, :]
    rxy_sc = rxy[:, :, :_TSC]
    xf, yf, w, tbl = _prep(q_sc, fm, rxy_sc, wv, wcat, bcat, B, _TSC, D, C)
    v4 = tbl.reshape(B * _H, 24, 24, _DH)
    vp = jnp.pad(v4, ((0, 0), (1, 2), (1, 2), (0, 0)))
    lo = lax.bitcast_convert_type(
        vp[:, :, 0:26, :].astype(jnp.bfloat16), jnp.uint16).astype(jnp.uint32)
    hi = lax.bitcast_convert_type(
        vp[:, :, 1:27, :].astype(jnp.bfloat16), jnp.uint16).astype(jnp.uint32)
    packed = lax.bitcast_convert_type(lo | (hi << 16), jnp.int32)
    tblp = jnp.pad(packed, ((0, 0), (0, 0), (0, 0), (0, 1))
                   ).reshape(B * _H, 27 * 26 * 33)
    ctx = _sc_sample(xf, yf, w, tblp, B * _H, _TSC)
    out_sc = _outproj(ctx, wout, bout, B, _TSC, D)
    woff = W_off.reshape(_H, 2 * _P, D)
    boff = b_off.reshape(_H, 2 * _P, 1)
    ww2 = W_w.reshape(_H, _P, D)
    bw2 = b_w.reshape(_H, _P, 1)
    out_tc = _tent(q[:, _TSC:, :], fm, rxy[:, :, _TSC:], wv, woff, boff,
                   ww2, bw2, wout, bout, B, T - _TSC, D, C)
    return jnp.concatenate([out_sc, out_tc], axis=1)


# speedup vs baseline: 1.1117x; 1.1117x over previous
import jax
import jax.numpy as jnp
from jax import lax
from jax.experimental import pallas as pl
from jax.experimental.pallas import tpu as pltpu
from jax.experimental.pallas import tpu_sc as plsc

_H = 8
_P = 16
_R = 0.08
_G = 576
_DH = 32
_CH = 256
_TT2 = 512
_TSC = 2048


def _dot(a, b, dims):
    return jax.lax.dot_general(a, b, (dims, ((), ())),
                               preferred_element_type=jnp.float32)


def _prep_body(q_ref, fm_ref, rxy_ref, wv_ref, wcat_ref, bcat_ref,
               xf_ref, yf_ref, w_ref, tbl_ref):
    q = q_ref[0]
    proj = _dot(wcat_ref[0], q, ((1,), (1,))) + bcat_ref[0]
    offx = proj[0:16, :]
    offy = proj[16:32, :]
    wl = proj[32:48, :]
    wl = wl - jnp.max(wl, axis=0, keepdims=True)
    we = jnp.exp(wl)
    w_ref[0] = we / jnp.sum(we, axis=0, keepdims=True)
    rx = rxy_ref[0, 0:1, :]
    ry = rxy_ref[0, 1:2, :]
    xf_ref[0] = (rx + _R * offx) * 23.0
    yf_ref[0] = (ry + _R * offy) * 23.0
    tbl_ref[0] = _dot(fm_ref[0], wv_ref[0], ((0,), (1,)))


def _prep(q, fm, rxy, wv, wcat, bcat, B, T, D, C):
    return pl.pallas_call(
        _prep_body,
        grid=(B, _H),
        in_specs=[
            pl.BlockSpec((1, T, D), lambda b, h: (b, 0, 0)),
            pl.BlockSpec((1, C, _G), lambda b, h: (b, 0, 0)),
            pl.BlockSpec((1, 2, T), lambda b, h: (b, 0, 0)),
            pl.BlockSpec((1, _DH, C), lambda b, h: (h, 0, 0)),
            pl.BlockSpec((1, 48, D), lambda b, h: (h, 0, 0)),
            pl.BlockSpec((1, 48, 1), lambda b, h: (h, 0, 0)),
        ],
        out_specs=[
            pl.BlockSpec((1, _P, T), lambda b, h: (b * _H + h, 0, 0)),
            pl.BlockSpec((1, _P, T), lambda b, h: (b * _H + h, 0, 0)),
            pl.BlockSpec((1, _P, T), lambda b, h: (b * _H + h, 0, 0)),
            pl.BlockSpec((1, _G, _DH), lambda b, h: (b * _H + h, 0, 0)),
        ],
        out_shape=[
            jax.ShapeDtypeStruct((B * _H, _P, T), jnp.float32),
            jax.ShapeDtypeStruct((B * _H, _P, T), jnp.float32),
            jax.ShapeDtypeStruct((B * _H, _P, T), jnp.float32),
            jax.ShapeDtypeStruct((B * _H, _G, _DH), jnp.float32),
        ],
    )(q, fm, rxy, wv, wcat, bcat)


def _sc_body(xf_hbm, yf_hbm, w_hbm, tbl_hbm, out_hbm,
             tblb, xfb, yfb, wb, outb):
    wid = lax.axis_index("s") * 2 + lax.axis_index("c")
    pltpu.sync_copy(tbl_hbm.at[wid], tblb)

    def chunk_body(i, _):
        base = i * _CH
        pltpu.sync_copy(xf_hbm.at[wid, :, pl.ds(base, _CH)], xfb)
        pltpu.sync_copy(yf_hbm.at[wid, :, pl.ds(base, _CH)], yfb)
        pltpu.sync_copy(w_hbm.at[wid, :, pl.ds(base, _CH)], wb)

        def g_body(g, _):
            sl = pl.ds(g * 16, 16)
            for blk in range(_DH // 16):
                zeros16 = tuple(jnp.zeros((16,), jnp.float32) for _ in range(16))

                @plsc.parallel_loop(0, _P, carry=zeros16)
                def p_body(p, acc):
                    xfv = xfb[p, sl]
                    yfv = yfb[p, sl]
                    wv = wb[p, sl]
                    xt = xfv.astype(jnp.int32)
                    yt = yfv.astype(jnp.int32)
                    x0 = xt - jnp.where(xt.astype(jnp.float32) > xfv, 1, 0)
                    y0 = yt - jnp.where(yt.astype(jnp.float32) > yfv, 1, 0)
                    valid = ((xfv > -1.0) & (xfv < 24.0)
                             & (yfv > -1.0) & (yfv < 24.0))
                    wv = jnp.where(valid, wv, jnp.zeros((16,), jnp.float32))
                    x0 = jnp.clip(x0, -1, 24)
                    y0 = jnp.clip(y0, -1, 24)
                    fx = xfv - x0.astype(jnp.float32)
                    fy = yfv - y0.astype(jnp.float32)
                    wy1 = wv * fy
                    wy0 = wv - wy1
                    c01 = wy0 * fx
                    c00 = wy0 - c01
                    c11 = wy1 * fx
                    c10 = wy1 - c11
                    a0 = ((y0 + 1) * 26 + (x0 + 1)) * 33 + (blk * 16)
                    new = []
                    for c in range(16):
                        r0 = plsc.load_gather(tblb, [a0 + c])
                        r1 = plsc.load_gather(tblb, [a0 + (26 * 33 + c)])
                        lo0 = plsc.bitcast(r0 << 16, jnp.float32)
                        hi0 = plsc.bitcast(r0, jnp.float32)
                        lo1 = plsc.bitcast(r1 << 16, jnp.float32)
                        hi1 = plsc.bitcast(r1, jnp.float32)
                        v = (c00 * lo0 + c01 * hi0) + (c10 * lo1 + c11 * hi1)
                        new.append(acc[c] + v)
                    return tuple(new)

                for c in range(16):
                    outb[blk * 16 + c, sl] = p_body[c]
            return 0

        lax.fori_loop(0, _CH // 16, g_body, 0)
        pltpu.sync_copy(outb, out_hbm.at[wid, :, pl.ds(base, _CH)])
        return 0

    lax.fori_loop(0, _TSC // _CH, chunk_body, 0)


def _sc_sample(xf, yf, w, tbl, BH, T):
    mesh = plsc.VectorSubcoreMesh(core_axis_name="c", subcore_axis_name="s")
    return pl.kernel(
        _sc_body,
        out_type=jax.ShapeDtypeStruct((BH, _DH, T), jnp.float32),
        mesh=mesh,
        compiler_params=pltpu.CompilerParams(needs_layout_passes=False),
        scratch_types=[
            pltpu.VMEM((27 * 26 * 33,), jnp.int32),
            pltpu.VMEM((_P, _CH), jnp.float32),
            pltpu.VMEM((_P, _CH), jnp.float32),
            pltpu.VMEM((_P, _CH), jnp.float32),
            pltpu.VMEM((_DH, _CH), jnp.float32),
        ],
    )(xf, yf, w, tbl)


def _out_body(ctx_ref, wout_ref, bout_ref, out_ref):
    h = pl.program_id(2)
    contrib = _dot(ctx_ref[0], wout_ref[0], ((0,), (0,)))

    @pl.when(h == 0)
    def _():
        out_ref[0] = contrib + bout_ref[...]

    @pl.when(h != 0)
    def _():
        out_ref[0] += contrib


def _outproj(ctx, wout, bout, B, T, D):
    return pl.pallas_call(
        _out_body,
        grid=(B, T // _TT2, _H),
        in_specs=[
            pl.BlockSpec((1, _DH, _TT2), lambda b, t, h: (b * _H + h, 0, t)),
            pl.BlockSpec((1, _DH, D), lambda b, t, h: (h, 0, 0)),
            pl.BlockSpec((1, D), lambda b, t, h: (0, 0)),
        ],
        out_specs=pl.BlockSpec((1, _TT2, D), lambda b, t, h: (b, t, 0)),
        out_shape=jax.ShapeDtypeStruct((B, T, D), jnp.float32),
    )(ctx, wout, bout)


_TTC = 2048


def _tent_body(q_ref, fm_ref, rxy_ref, wv_ref, woff_ref, boff_ref, ww_ref,
               bw_ref, wout_ref, bout_ref, out_ref, vs_ref):
    t_idx = pl.program_id(1)
    h = pl.program_id(2)

    @pl.when(t_idx == 0)
    def _():
        vs_ref[h] = _dot(wv_ref[0], fm_ref[0], ((1,), (0,)))

    q = q_ref[0]
    off = _dot(woff_ref[0], q, ((1,), (1,))) + boff_ref[0]
    wl = _dot(ww_ref[0], q, ((1,), (1,))) + bw_ref[0]
    wl = wl - jnp.max(wl, axis=0, keepdims=True)
    we = jnp.exp(wl)
    w = we / jnp.sum(we, axis=0, keepdims=True)

    rx = rxy_ref[0, 0:1, :]
    ry = rxy_ref[0, 1:2, :]
    xs = jax.lax.broadcasted_iota(jnp.int32, (24, 1), 0).astype(jnp.float32)

    acc = jnp.zeros((24, 24, _TTC), jnp.float32)
    for p in range(_P):
        xf = (rx + _R * off[2 * p:2 * p + 1, :]) * 23.0
        yf = (ry + _R * off[2 * p + 1:2 * p + 2, :]) * 23.0
        tx = jnp.maximum(1.0 - jnp.abs(xs - xf), 0.0)
        ty = jnp.maximum(1.0 - jnp.abs(xs - yf), 0.0)
        wtx = w[p:p + 1, :] * tx
        acc = acc + wtx[None, :, :] * ty[:, None, :]
    acc = acc.reshape(_G, _TTC)

    ctx = _dot(vs_ref[h], acc, ((1,), (0,)))
    contrib = _dot(ctx, wout_ref[0], ((0,), (0,)))

    @pl.when(h == 0)
    def _():
        out_ref[0] = contrib + bout_ref[...]

    @pl.when(h != 0)
    def _():
        out_ref[0] += contrib


def _tent(q2, fm, rxy2, wv, woff, boff, ww, bw, wout, bout, B, T2, D, C):
    return pl.pallas_call(
        _tent_body,
        grid=(B, T2 // _TTC, _H),
        in_specs=[
            pl.BlockSpec((1, _TTC, D), lambda b, t, h: (b, t, 0)),
            pl.BlockSpec((1, C, _G), lambda b, t, h: (b, 0, 0)),
            pl.BlockSpec((1, 2, _TTC), lambda b, t, h: (b, 0, t)),
            pl.BlockSpec((1, _DH, C), lambda b, t, h: (h, 0, 0)),
            pl.BlockSpec((1, 2 * _P, D), lambda b, t, h: (h, 0, 0)),
            pl.BlockSpec((1, 2 * _P, 1), lambda b, t, h: (h, 0, 0)),
            pl.BlockSpec((1, _P, D), lambda b, t, h: (h, 0, 0)),
            pl.BlockSpec((1, _P, 1), lambda b, t, h: (h, 0, 0)),
            pl.BlockSpec((1, _DH, D), lambda b, t, h: (h, 0, 0)),
            pl.BlockSpec((1, D), lambda b, t, h: (0, 0)),
        ],
        out_specs=pl.BlockSpec((1, _TTC, D), lambda b, t, h: (b, t, 0)),
        out_shape=jax.ShapeDtypeStruct((B, T2, D), jnp.float32),
        scratch_shapes=[pltpu.VMEM((_H, _DH, _G), jnp.float32)],
    )(q2, fm, rxy2, wv, woff, boff, ww, bw, wout, bout)


def kernel(q, fmap, ref_xy, W_v, W_off, b_off, W_w, b_w, W_out, b_out):
    B, T, D = q.shape
    C = fmap.shape[1]
    fm = fmap.reshape(B, C, _G)
    rxy = ref_xy.transpose(0, 2, 1)
    wv = W_v.reshape(_H, _DH, C)
    woff4 = W_off.reshape(_H, _P, 2, D)
    wcat = jnp.concatenate(
        [woff4[:, :, 0, :], woff4[:, :, 1, :], W_w.reshape(_H, _P, D)], axis=1)
    boff4 = b_off.reshape(_H, _P, 2)
    bcat = jnp.concatenate(
        [boff4[:, :, 0], boff4[:, :, 1], b_w.reshape(_H, _P)],
        axis=1).reshape(_H, 48, 1)
    wout = W_out.T.reshape(_H, _DH, D)
    bout = b_out.reshape(1, D)

    q_sc = q[:, :_TSC, :]
    rxy_sc = rxy[:, :, :_TSC]
    xf, yf, w, tbl = _prep(q_sc, fm, rxy_sc, wv, wcat, bcat, B, _TSC, D, C)
    v4 = tbl.reshape(B * _H, 24, 24, _DH)
    vp = jnp.pad(v4, ((0, 0), (1, 2), (1, 2), (0, 0)))
    lo = lax.bitcast_convert_type(
        vp[:, :, 0:26, :].astype(jnp.bfloat16), jnp.uint16).astype(jnp.uint32)
    hi = lax.bitcast_convert_type(
        vp[:, :, 1:27, :].astype(jnp.bfloat16), jnp.uint16).astype(jnp.uint32)
    packed = lax.bitcast_convert_type(lo | (hi << 16), jnp.int32)
    tblp = jnp.pad(packed, ((0, 0), (0, 0), (0, 0), (0, 1))
                   ).reshape(B * _H, 27 * 26 * 33)
    ctx = _sc_sample(xf, yf, w, tblp, B * _H, _TSC)
    out_sc = _outproj(ctx, wout, bout, B, _TSC, D)
    woff = W_off.reshape(_H, 2 * _P, D)
    boff = b_off.reshape(_H, 2 * _P, 1)
    ww2 = W_w.reshape(_H, _P, D)
    bw2 = b_w.reshape(_H, _P, 1)
    out_tc = _tent(q[:, _TSC:, :], fm, rxy[:, :, _TSC:], wv, woff, boff,
                   ww2, bw2, wout, bout, B, T - _TSC, D, C)
    return jnp.concatenate([out_sc, out_tc], axis=1)
